# baseline (speedup 1.0000x reference)
import jax
import jax.numpy as jnp
from jax.experimental import pallas as pl
from jax.experimental.pallas import tpu as pltpu

_NUM_PRE = 2048
_NUM_POST = 1024
_BATCH = 16
_K_TARGET = int(0.1 * _NUM_PRE * _NUM_POST)
_DT = 0.1
_BLK = 128
_G = _NUM_PRE // _BLK


def _body(scal_ref, pre_blk_ref, post_ref, elig_blk_ref, wu_ref, elig_out_ref,
          ne_scr, bits_scr, smem):
    i = pl.program_id(0)
    decay = scal_ref[0]
    scale = scal_ref[1]
    thr = scal_ref[2]
    mwc = scal_ref[3]

    @pl.when(i < _G)
    def _phase_a():
        upd = jnp.dot(pre_blk_ref[...], post_ref[...],
                      preferred_element_type=jnp.float32) * scale
        ne = elig_blk_ref[...] * decay + upd
        elig_out_ref[...] = ne
        act = jnp.abs(ne)
        ne_scr[pl.ds(i * _BLK, _BLK), :] = ne
        bits_scr[pl.ds(i * _BLK, _BLK), :] = jax.lax.bitcast_convert_type(
            act, jnp.int32)
        cnt = jnp.sum((act > thr).astype(jnp.int32))
        prev = jnp.where(i == 0, jnp.int32(0), smem[0])
        smem[0] = prev + cnt

    @pl.when(i == _G)
    def _phase_b():
        bits = bits_scr[...]

        def search_step(_, lohi):
            lo, hi = lohi
            mid = lo + (hi - lo) // 2
            c = jnp.sum((bits >= mid).astype(jnp.int32))
            ge = c >= _K_TARGET
            return jnp.where(ge, mid, lo), jnp.where(ge, hi, mid)

        lo0 = jnp.int32(0)
        hi0 = jnp.int32(0x7F800000)
        tstar, _ = jax.lax.fori_loop(0, 31, search_step, (lo0, hi0))

        use_topk = smem[0] > _K_TARGET
        thr_bits = jax.lax.bitcast_convert_type(thr, jnp.int32)
        thr_cut = jnp.where(thr >= 0.0, thr_bits + 1, jnp.int32(0))
        smem[1] = jnp.where(use_topk, tstar, thr_cut)

    @pl.when(i >= _G)
    def _phase_c():
        j = i - _G
        cut = smem[1]
        ne = ne_scr[pl.ds(j * _BLK, _BLK), :]
        abits = bits_scr[pl.ds(j * _BLK, _BLK), :]
        mask = abits >= cut
        wu_ref[...] = jnp.where(mask, jnp.clip(ne, -mwc, mwc),
                                jnp.zeros_like(ne))


def kernel(pre_spikes, post_spikes, weights, eligibility_trace, a_plus,
           a_minus, tau_plus, tau_minus, tau_eligibility, activity_threshold,
           max_weight_change):
    del weights, tau_plus, tau_minus
    decay = jnp.exp(-_DT / tau_eligibility)
    scale = (a_plus + a_minus) / _BATCH
    scalars = jnp.stack([decay, scale, activity_threshold,
                         max_weight_change]).astype(jnp.float32)
    pre_t = pre_spikes.T

    out_shape = (
        jax.ShapeDtypeStruct((_NUM_PRE, _NUM_POST), jnp.float32),
        jax.ShapeDtypeStruct((_NUM_PRE, _NUM_POST), jnp.float32),
    )
    wu, new_elig = pl.pallas_call(
        _body,
        grid=(2 * _G,),
        out_shape=out_shape,
        in_specs=[
            pl.BlockSpec(memory_space=pltpu.SMEM),
            pl.BlockSpec((_BLK, _BATCH),
                         lambda i: (jnp.minimum(i, _G - 1), 0)),
            pl.BlockSpec((_BATCH, _NUM_POST), lambda i: (0, 0)),
            pl.BlockSpec((_BLK, _NUM_POST),
                         lambda i: (jnp.minimum(i, _G - 1), 0)),
        ],
        out_specs=(
            pl.BlockSpec((_BLK, _NUM_POST),
                         lambda i: (jnp.maximum(i - _G, 0), 0)),
            pl.BlockSpec((_BLK, _NUM_POST),
                         lambda i: (jnp.minimum(i, _G - 1), 0)),
        ),
        scratch_shapes=[
            pltpu.VMEM((_NUM_PRE, _NUM_POST), jnp.float32),
            pltpu.VMEM((_NUM_PRE, _NUM_POST), jnp.int32),
            pltpu.SMEM((2,), jnp.int32),
        ],
        compiler_params=pltpu.CompilerParams(
            dimension_semantics=("arbitrary",)),
    )(scalars, pre_t, post_spikes, eligibility_trace)
    return (wu, new_elig)


# speedup vs baseline: 1.4127x; 1.4127x over previous
import jax
import jax.numpy as jnp
from jax.experimental import pallas as pl
from jax.experimental.pallas import tpu as pltpu

_NUM_PRE = 2048
_NUM_POST = 1024
_BATCH = 16
_N = _NUM_PRE * _NUM_POST
_K_TARGET = int(0.1 * _N)
_DT = 0.1
_BLK = 256
_G = _NUM_PRE // _BLK

_SUB_ROWS = 128
_K_SUB = _K_TARGET // 16
_M_SUB = 656
_RANK_TOL = 3
_INF_BITS = 0x7F800000


def _bisect(data, target, n_iter):
    def step(_, lohi):
        lo, hi = lohi
        mid = lo + (hi - lo) // 2
        c = jnp.sum((data >= mid).astype(jnp.int32))
        ge = c >= target
        return jnp.where(ge, mid, lo), jnp.where(ge, hi, mid)

    lo, _ = jax.lax.fori_loop(0, n_iter, step, (jnp.int32(0),
                                                jnp.int32(_INF_BITS)))
    return lo


def _body(scal_ref, pre_blk_ref, post_ref, elig_blk_ref, wu_ref, elig_out_ref,
          ne_scr, bits_scr, smem):
    i = pl.program_id(0)
    decay = scal_ref[0]
    scale = scal_ref[1]
    thr = scal_ref[2]
    mwc = scal_ref[3]

    @pl.when(i < _G)
    def _phase_a():
        upd = jnp.dot(pre_blk_ref[...], post_ref[...],
                      preferred_element_type=jnp.float32) * scale
        ne = elig_blk_ref[...] * decay + upd
        elig_out_ref[...] = ne
        act = jnp.abs(ne)
        ne_scr[pl.ds(i * _BLK, _BLK), :] = ne
        bits_scr[pl.ds(i * _BLK, _BLK), :] = jax.lax.bitcast_convert_type(
            act, jnp.int32)
        cnt = jnp.sum((act > thr).astype(jnp.int32))
        prev = jnp.where(i == 0, jnp.int32(0), smem[0])
        smem[0] = prev + cnt

    @pl.when(i == _G)
    def _phase_b():
        sub = bits_scr[0:_SUB_ROWS, :]
        t_lo = _bisect(sub, _K_SUB + _M_SUB, 31)
        t_hi = _bisect(sub, _K_SUB - _M_SUB, 31)

        bits = bits_scr[...]

        def cond(state):
            it, lo, hi, c_lo = state
            return jnp.logical_and(c_lo > _K_TARGET + _RANK_TOL, hi - lo > 1)

        def body(state):
            it, lo, hi, c_lo = state
            mid = lo + (hi - lo) // 2
            mid = jnp.where(it == 0, t_lo, jnp.where(it == 1, t_hi, mid))
            mid = jnp.clip(mid, lo + 1, jnp.maximum(hi - 1, lo + 1))
            c = jnp.sum((bits >= mid).astype(jnp.int32))
            ge = c >= _K_TARGET
            return (it + 1,
                    jnp.where(ge, mid, lo),
                    jnp.where(ge, hi, mid),
                    jnp.where(ge, c, c_lo))

        _, tstar, _, _ = jax.lax.while_loop(
            cond, body,
            (jnp.int32(0), jnp.int32(0), jnp.int32(_INF_BITS), jnp.int32(_N)))

        use_topk = smem[0] > _K_TARGET
        thr_bits = jax.lax.bitcast_convert_type(thr, jnp.int32)
        thr_cut = jnp.where(thr >= 0.0, thr_bits + 1, jnp.int32(0))
        smem[1] = jnp.where(use_topk, tstar, thr_cut)

    @pl.when(i >= _G)
    def _phase_c():
        j = i - _G
        cut = smem[1]
        ne = ne_scr[pl.ds(j * _BLK, _BLK), :]
        abits = bits_scr[pl.ds(j * _BLK, _BLK), :]
        mask = abits >= cut
        wu_ref[...] = jnp.where(mask, jnp.clip(ne, -mwc, mwc),
                                jnp.zeros_like(ne))


def kernel(pre_spikes, post_spikes, weights, eligibility_trace, a_plus,
           a_minus, tau_plus, tau_minus, tau_eligibility, activity_threshold,
           max_weight_change):
    del weights, tau_plus, tau_minus
    decay = jnp.exp(-_DT / tau_eligibility)
    scale = (a_plus + a_minus) / _BATCH
    scalars = jnp.stack([decay, scale, activity_threshold,
                         max_weight_change]).astype(jnp.float32)
    pre_t = pre_spikes.T

    out_shape = (
        jax.ShapeDtypeStruct((_NUM_PRE, _NUM_POST), jnp.float32),
        jax.ShapeDtypeStruct((_NUM_PRE, _NUM_POST), jnp.float32),
    )
    wu, new_elig = pl.pallas_call(
        _body,
        grid=(2 * _G,),
        out_shape=out_shape,
        in_specs=[
            pl.BlockSpec(memory_space=pltpu.SMEM),
            pl.BlockSpec((_BLK, _BATCH),
                         lambda i: (jnp.minimum(i, _G - 1), 0)),
            pl.BlockSpec((_BATCH, _NUM_POST), lambda i: (0, 0)),
            pl.BlockSpec((_BLK, _NUM_POST),
                         lambda i: (jnp.minimum(i, _G - 1), 0)),
        ],
        out_specs=(
            pl.BlockSpec((_BLK, _NUM_POST),
                         lambda i: (jnp.maximum(i - _G, 0), 0)),
            pl.BlockSpec((_BLK, _NUM_POST),
                         lambda i: (jnp.minimum(i, _G - 1), 0)),
        ),
        scratch_shapes=[
            pltpu.VMEM((_NUM_PRE, _NUM_POST), jnp.float32),
            pltpu.VMEM((_NUM_PRE, _NUM_POST), jnp.int32),
            pltpu.SMEM((2,), jnp.int32),
        ],
        compiler_params=pltpu.CompilerParams(
            dimension_semantics=("arbitrary",)),
    )(scalars, pre_t, post_spikes, eligibility_trace)
    return (wu, new_elig)
